# baseline (speedup 1.0000x reference)
import jax
import jax.numpy as jnp
from jax.experimental import pallas as pl

IMAGE_H = 400
IMAGE_W = 400
N_RAYS = 1024
N_PTS = 64
MIN_DEPTH = 0.1
MAX_DEPTH = 10.0


def kernel(mask, R, T):
    B = mask.shape[0]
    probs = mask.reshape(B, -1)
    probs = probs / jnp.clip(jnp.sum(probs, axis=-1, keepdims=True), 1e-12)
    cdf = jnp.cumsum(probs, axis=-1)
    u = jax.random.uniform(jax.random.key(42), (B, N_RAYS), dtype=jnp.float32)
    idx = jax.vmap(lambda c, uu: jnp.searchsorted(c, uu))(cdf, u)
    idx = jnp.clip(idx, 0, probs.shape[-1] - 1)
    ys = idx // IMAGE_W
    xs = idx % IMAGE_W
    x_ndc = 1.0 - 2.0 * (xs.astype(jnp.float32) + 0.5) / IMAGE_W
    y_ndc = 1.0 - 2.0 * (ys.astype(jnp.float32) + 0.5) / IMAGE_H
    xys = jnp.stack([x_ndc, y_ndc], axis=-1)
    dirs_cam = jnp.stack([x_ndc, y_ndc, jnp.ones_like(x_ndc)], axis=-1)
    dirs_world = jnp.einsum('bri,bij->brj', dirs_cam, R)
    dirs_world = dirs_world / jnp.linalg.norm(dirs_world, axis=-1, keepdims=True)
    centers = -jnp.einsum('bij,bj->bi', R, T)
    origins = jnp.broadcast_to(centers[:, None, :], (B, N_RAYS, 3))
    edges = jnp.linspace(MIN_DEPTH, MAX_DEPTH, N_PTS + 1, dtype=jnp.float32)
    lower, upper = edges[:-1], edges[1:]
    jitter = jax.random.uniform(jax.random.key(7), (B, N_RAYS, N_PTS), dtype=jnp.float32)
    lengths = lower + (upper - lower) * jitter
    return origins, dirs_world, lengths, xys
